# baseline (speedup 1.0000x reference)
import functools

import jax
import jax.numpy as jnp
from jax import lax
from jax.experimental import pallas as pl
from jax.experimental.pallas import tpu as pltpu
from jax.experimental.pallas import tpu_sc as plsc

N_EMB = 8192
DIM = 256
C_WEIGHT = 0.25

TM = 4096

NW = 32
CH = 128

WINDOWS = ((0, 2736), (2736, 2736), (5472, 2720))
CHUNK = 1024
LG = 128


def _group_window(gstart):
    out = []
    for w, (ws, sz) in enumerate(WINDOWS):
        lo = max(gstart, ws)
        hi = min(gstart + LG, ws + sz)
        if lo < hi:
            out.append((w, lo - gstart, hi - gstart))
    return out


def _argmin_body(z_ref, zsq_ref, emb_ref, esq_ref, idx_ref, mind_ref):
    z = z_ref[...]
    zsq = zsq_ref[...]
    z2 = z + z

    lane = lax.broadcasted_iota(jnp.int32, (TM, LG), 1).astype(jnp.float32)
    inf_g = jnp.full((TM, LG), jnp.inf, jnp.float32)

    M = [inf_g for _ in WINDOWS]
    C = [jnp.zeros((TM, LG), jnp.float32) for _ in WINDOWS]

    for cstart in range(0, N_EMB, CHUNK):
        e = emb_ref[pl.ds(cstart, CHUNK), :]
        esq = esq_ref[pl.ds(cstart, CHUNK)]
        ze2 = lax.dot_general(z2, e, (((1,), (1,)), ((), ())),
                              preferred_element_type=jnp.float32)
        d = (zsq + esq[None, :]) - ze2
        for g in range(CHUNK // LG):
            gstart = cstart + g * LG
            dg = d[:, g * LG:(g + 1) * LG]
            gc = float(gstart // LG)
            for (w, llo, lhi) in _group_window(gstart):
                if llo == 0 and lhi == LG:
                    dgw = dg
                else:
                    sel = (lane >= float(llo)) & (lane < float(lhi))
                    dgw = jnp.where(sel, dg, inf_g)
                better = dgw < M[w]
                C[w] = jnp.where(better, gc, C[w])
                M[w] = jnp.where(better, dgw, M[w])

    acc_v16 = jnp.full((TM,), jnp.inf, jnp.bfloat16)
    acc_i = jnp.zeros((TM,), jnp.float32)
    gmin = jnp.full((TM,), jnp.inf, jnp.float32)
    for w in range(len(WINDOWS)):
        m = jnp.min(M[w], axis=1)
        jvec = C[w] * float(LG) + lane
        cand = jnp.where(M[w] == m[:, None], jvec, jnp.inf)
        a = jnp.min(cand, axis=1)
        av = acc_v16.astype(jnp.float32)
        keep_v = av < m
        keep_i = keep_v | (av == m)
        acc_i = jnp.where(keep_i, acc_i, a)
        acc_v16 = jnp.where(keep_v, av, m).astype(jnp.bfloat16)
        gmin = jnp.minimum(gmin, m)
    idx_ref[...] = acc_i.astype(jnp.int32)
    mind_ref[...] = gmin


def _tc_argmin(z_flat, z_sq, embeddings, e_sq):
    m = z_flat.shape[0]
    grid = (m // TM,)
    return pl.pallas_call(
        _argmin_body,
        grid=grid,
        in_specs=[
            pl.BlockSpec((TM, DIM), lambda i: (i, 0)),
            pl.BlockSpec((TM, 1), lambda i: (i, 0)),
            pl.BlockSpec((N_EMB, DIM), lambda i: (0, 0)),
            pl.BlockSpec((N_EMB,), lambda i: (0,)),
        ],
        out_specs=[
            pl.BlockSpec((TM,), lambda i: (i,)),
            pl.BlockSpec((TM,), lambda i: (i,)),
        ],
        out_shape=[
            jax.ShapeDtypeStruct((m,), jnp.int32),
            jax.ShapeDtypeStruct((m,), jnp.float32),
        ],
        compiler_params=pltpu.CompilerParams(
            dimension_semantics=("arbitrary",),
        ),
    )(z_flat, z_sq, embeddings, e_sq)


def _sc_gather(embeddings, idx3):
    n_chunks = idx3.shape[1]
    rows_per_w = n_chunks * CH
    b = NW * rows_per_w
    mesh = plsc.VectorSubcoreMesh(core_axis_name="c", subcore_axis_name="s")

    @functools.partial(
        pl.kernel,
        mesh=mesh,
        out_type=jax.ShapeDtypeStruct((b, DIM), jnp.float32),
        scratch_types=[
            pltpu.VMEM((n_chunks, CH), jnp.int32),
            pltpu.VMEM((CH, DIM), jnp.float32),
            pltpu.VMEM((CH, DIM), jnp.float32),
            pltpu.SemaphoreType.DMA,
            pltpu.SemaphoreType.DMA,
        ],
    )
    def gather(emb_hbm, idx_hbm, out_hbm, idx_v, rows0, rows1, sem0, sem1):
        wid = lax.axis_index("s") * 2 + lax.axis_index("c")
        base = wid * rows_per_w
        pltpu.sync_copy(idx_hbm.at[wid], idx_v)

        bufs = (rows0, rows1)
        sems = (sem0, sem1)
        copies = []
        for c in range(n_chunks):
            cp = pltpu.async_copy(emb_hbm.at[idx_v.at[c]], bufs[c % 2], sems[c % 2])
            copies.append(cp)
            if c >= 1:
                copies[c - 1].wait()
                pltpu.sync_copy(bufs[(c - 1) % 2],
                                out_hbm.at[pl.ds(base + (c - 1) * CH, CH)])
        copies[-1].wait()
        pltpu.sync_copy(bufs[(n_chunks - 1) % 2],
                        out_hbm.at[pl.ds(base + (n_chunks - 1) * CH, CH)])

    return gather(embeddings, idx3)


def kernel(z, embeddings):
    input_shape = z.shape
    z_flat = z.reshape(-1, DIM)
    m = z_flat.shape[0]
    z_sq = jnp.sum(z_flat ** 2, axis=1, keepdims=True)
    e_sq = jnp.sum(embeddings ** 2, axis=1)

    idx_flat, min_d = _tc_argmin(z_flat, z_sq, embeddings, e_sq)

    idx3 = idx_flat.reshape(NW, m // (NW * CH), CH)
    z_q_flat = _sc_gather(embeddings, idx3)

    z_q = z_q_flat.reshape(input_shape)
    idx = idx_flat.reshape(input_shape[:-1])
    z_q_st = z + (z_q - z)
    loss = jnp.sum(min_d) * (C_WEIGHT / (m * DIM))
    return (z_q_st, idx, loss)


# speedup vs baseline: 1.0695x; 1.0695x over previous
import functools

import jax
import jax.numpy as jnp
from jax import lax
from jax.experimental import pallas as pl
from jax.experimental.pallas import tpu as pltpu
from jax.experimental.pallas import tpu_sc as plsc

N_EMB = 8192
DIM = 256
C_WEIGHT = 0.25

TM = 2048

NW = 32
CH = 128

WINDOWS = ((0, 2736), (2736, 2736), (5472, 2720))
CHUNK = 1024
LG = 128


def _group_window(gstart):
    out = []
    for w, (ws, sz) in enumerate(WINDOWS):
        lo = max(gstart, ws)
        hi = min(gstart + LG, ws + sz)
        if lo < hi:
            out.append((w, lo - gstart, hi - gstart))
    return out


def _argmin_body(z_ref, zsq_ref, emb_ref, esq_ref, idx_ref, mind_ref):
    z = z_ref[...]
    zsq = zsq_ref[...]
    z2 = z + z

    lane = lax.broadcasted_iota(jnp.int32, (TM, LG), 1).astype(jnp.float32)
    inf_g = jnp.full((TM, LG), jnp.inf, jnp.float32)

    M = [inf_g for _ in WINDOWS]
    C = [jnp.zeros((TM, LG), jnp.float32) for _ in WINDOWS]

    for cstart in range(0, N_EMB, CHUNK):
        e = emb_ref[pl.ds(cstart, CHUNK), :]
        esq = esq_ref[pl.ds(cstart, CHUNK)]
        ze2 = lax.dot_general(z2, e, (((1,), (1,)), ((), ())),
                              preferred_element_type=jnp.float32)
        d = (zsq + esq[None, :]) - ze2
        for g in range(CHUNK // LG):
            gstart = cstart + g * LG
            dg = d[:, g * LG:(g + 1) * LG]
            gc = float(gstart // LG)
            for (w, llo, lhi) in _group_window(gstart):
                if llo == 0 and lhi == LG:
                    dgw = dg
                else:
                    sel = (lane >= float(llo)) & (lane < float(lhi))
                    dgw = jnp.where(sel, dg, inf_g)
                better = dgw < M[w]
                C[w] = jnp.where(better, gc, C[w])
                M[w] = jnp.where(better, dgw, M[w])

    acc_v16 = jnp.full((TM,), jnp.inf, jnp.bfloat16)
    acc_i = jnp.zeros((TM,), jnp.float32)
    gmin = jnp.full((TM,), jnp.inf, jnp.float32)
    for w in range(len(WINDOWS)):
        m = jnp.min(M[w], axis=1)
        jvec = C[w] * float(LG) + lane
        cand = jnp.where(M[w] == m[:, None], jvec, jnp.inf)
        a = jnp.min(cand, axis=1)
        av = acc_v16.astype(jnp.float32)
        keep_v = av < m
        keep_i = keep_v | (av == m)
        acc_i = jnp.where(keep_i, acc_i, a)
        acc_v16 = jnp.where(keep_v, av, m).astype(jnp.bfloat16)
        gmin = jnp.minimum(gmin, m)
    idx_ref[...] = acc_i.astype(jnp.int32)
    mind_ref[...] = gmin


def _tc_argmin(z_flat, z_sq, embeddings, e_sq):
    m = z_flat.shape[0]
    grid = (m // TM,)
    return pl.pallas_call(
        _argmin_body,
        grid=grid,
        in_specs=[
            pl.BlockSpec((TM, DIM), lambda i: (i, 0)),
            pl.BlockSpec((TM, 1), lambda i: (i, 0)),
            pl.BlockSpec((N_EMB, DIM), lambda i: (0, 0)),
            pl.BlockSpec((N_EMB,), lambda i: (0,)),
        ],
        out_specs=[
            pl.BlockSpec((TM,), lambda i: (i,)),
            pl.BlockSpec((TM,), lambda i: (i,)),
        ],
        out_shape=[
            jax.ShapeDtypeStruct((m,), jnp.int32),
            jax.ShapeDtypeStruct((m,), jnp.float32),
        ],
        compiler_params=pltpu.CompilerParams(
            dimension_semantics=("arbitrary",),
        ),
    )(z_flat, z_sq, embeddings, e_sq)


def _sc_gather(embeddings, idx3):
    n_chunks = idx3.shape[1]
    rows_per_w = n_chunks * CH
    b = NW * rows_per_w
    mesh = plsc.VectorSubcoreMesh(core_axis_name="c", subcore_axis_name="s")

    @functools.partial(
        pl.kernel,
        mesh=mesh,
        out_type=jax.ShapeDtypeStruct((b, DIM), jnp.float32),
        scratch_types=[
            pltpu.VMEM((n_chunks, CH), jnp.int32),
            pltpu.VMEM((CH, DIM), jnp.float32),
            pltpu.VMEM((CH, DIM), jnp.float32),
            pltpu.SemaphoreType.DMA,
            pltpu.SemaphoreType.DMA,
        ],
    )
    def gather(emb_hbm, idx_hbm, out_hbm, idx_v, rows0, rows1, sem0, sem1):
        wid = lax.axis_index("s") * 2 + lax.axis_index("c")
        base = wid * rows_per_w
        pltpu.sync_copy(idx_hbm.at[wid], idx_v)

        bufs = (rows0, rows1)
        sems = (sem0, sem1)
        copies = []
        for c in range(n_chunks):
            cp = pltpu.async_copy(emb_hbm.at[idx_v.at[c]], bufs[c % 2], sems[c % 2])
            copies.append(cp)
            if c >= 1:
                copies[c - 1].wait()
                pltpu.sync_copy(bufs[(c - 1) % 2],
                                out_hbm.at[pl.ds(base + (c - 1) * CH, CH)])
        copies[-1].wait()
        pltpu.sync_copy(bufs[(n_chunks - 1) % 2],
                        out_hbm.at[pl.ds(base + (n_chunks - 1) * CH, CH)])

    return gather(embeddings, idx3)


def kernel(z, embeddings):
    input_shape = z.shape
    z_flat = z.reshape(-1, DIM)
    m = z_flat.shape[0]
    z_sq = jnp.sum(z_flat ** 2, axis=1, keepdims=True)
    e_sq = jnp.sum(embeddings ** 2, axis=1)

    idx_flat, min_d = _tc_argmin(z_flat, z_sq, embeddings, e_sq)

    idx3 = idx_flat.reshape(NW, m // (NW * CH), CH)
    z_q_flat = _sc_gather(embeddings, idx3)

    z_q = z_q_flat.reshape(input_shape)
    idx = idx_flat.reshape(input_shape[:-1])
    z_q_st = z + (z_q - z)
    loss = jnp.sum(min_d) * (C_WEIGHT / (m * DIM))
    return (z_q_st, idx, loss)
